# Initial kernel scaffold; baseline (speedup 1.0000x reference)
import jax
import jax.numpy as jnp
from jax.experimental import pallas as pl


def kernel(x, adj, W_cheb, b_cheb, W2, b2):
    raise NotImplementedError("write your pallas kernel here")


# speedup vs baseline: 1.0104x; 1.0104x over previous
import jax
import jax.numpy as jnp
from jax.experimental import pallas as pl
from jax.experimental.pallas import tpu as pltpu

N = 10000
NFEAT = 128
NHID = 16
NCLS = 2
R = 400
NBLK = N // R
NPASS = 4


def _body(adj_ref, x_ref, Wc_ref, bc_ref, W2p_ref, b2p_ref, out_ref,
          qs_ref, pd_ref, base_ref, dinv_ref, sc_ref, macc_ref):
    p = pl.program_id(0)
    i = pl.program_id(1)
    sl = pl.ds(i * R, R)

    @pl.when(p == 0)
    def _pass0():
        deg = jnp.sum(adj_ref[...], axis=1, keepdims=True)
        dinv = jnp.where(deg > 0.0,
                         jax.lax.rsqrt(jnp.maximum(deg, 1e-12)), 0.0)
        xb = x_ref[...]
        W0 = Wc_ref[0]
        W1 = Wc_ref[1]
        W2c = Wc_ref[2]
        P = jnp.dot(xb, W1, preferred_element_type=jnp.float32)
        Q = jnp.dot(xb, W2c, preferred_element_type=jnp.float32)
        base = jnp.dot(xb, W0 - W2c, preferred_element_type=jnp.float32)
        qs_ref[sl, :] = dinv * Q
        pd_ref[sl, :] = dinv * P
        base_ref[sl, :] = base + bc_ref[...]
        dinv_ref[sl, :] = jnp.broadcast_to(dinv, (R, NHID))

    @pl.when(p == 1)
    def _pass1():
        U = jnp.dot(adj_ref[...], qs_ref[...],
                    preferred_element_type=jnp.float32)
        dinv = dinv_ref[sl, :]
        sc_ref[sl, :] = 2.0 * (dinv * dinv) * U - pd_ref[sl, :]

    @pl.when(p == 2)
    def _pass2():
        T = jnp.dot(adj_ref[...], sc_ref[...],
                    preferred_element_type=jnp.float32)
        h = base_ref[sl, :] + dinv_ref[sl, :] * T
        h = jnp.maximum(h, 0.0)
        qs_ref[sl, :] = jnp.dot(h, W2p_ref[...],
                                preferred_element_type=jnp.float32)

    @pl.when(p == 3)
    def _pass3():
        O = jnp.dot(adj_ref[...], qs_ref[...],
                     preferred_element_type=jnp.float32)
        m = jnp.max(O, axis=0, keepdims=True)

        @pl.when(i == 0)
        def _():
            macc_ref[...] = m

        @pl.when(i > 0)
        def _():
            macc_ref[...] = jnp.maximum(macc_ref[...], m)

        @pl.when(i == NBLK - 1)
        def _():
            out_ref[...] = macc_ref[...] + b2p_ref[...]


def kernel(x, adj, W_cheb, b_cheb, W2, b2):
    bc2 = b_cheb.reshape(1, NHID)
    W2p = jnp.zeros((NHID, NHID), jnp.float32).at[:, :NCLS].set(W2)
    b2p = jnp.zeros((1, NHID), jnp.float32).at[0, :NCLS].set(b2)
    out = pl.pallas_call(
        _body,
        grid=(NPASS, NBLK),
        in_specs=[
            pl.BlockSpec((R, N), lambda p, i: (i, 0)),
            pl.BlockSpec((R, NFEAT), lambda p, i: (i, 0)),
            pl.BlockSpec((3, NFEAT, NHID), lambda p, i: (0, 0, 0)),
            pl.BlockSpec((1, NHID), lambda p, i: (0, 0)),
            pl.BlockSpec((NHID, NHID), lambda p, i: (0, 0)),
            pl.BlockSpec((1, NHID), lambda p, i: (0, 0)),
        ],
        out_specs=pl.BlockSpec((1, NHID), lambda p, i: (0, 0)),
        out_shape=jax.ShapeDtypeStruct((1, NHID), jnp.float32),
        scratch_shapes=[
            pltpu.VMEM((N, NHID), jnp.float32),
            pltpu.VMEM((N, NHID), jnp.float32),
            pltpu.VMEM((N, NHID), jnp.float32),
            pltpu.VMEM((N, NHID), jnp.float32),
            pltpu.VMEM((N, NHID), jnp.float32),
            pltpu.VMEM((1, NHID), jnp.float32),
        ],
        compiler_params=pltpu.CompilerParams(
            dimension_semantics=("arbitrary", "arbitrary"),
        ),
    )(adj, x, W_cheb, bc2, W2p, b2p)
    return out[:, :NCLS].reshape(1, 1, NCLS)
